# initial kernel scaffold (unmeasured)
import jax
import jax.numpy as jnp
from jax import lax
from jax.experimental import pallas as pl
from jax.experimental.pallas import tpu as pltpu

N_DEV = 32


def kernel(x, w_mat, scale_x, scale_w):
    m_per, k = x.shape
    _, n = w_mat.shape
    n_per = n // N_DEV

    def body(x_ref, w_ref, sx_ref, sw_ref, out_ref, y_ref, send_sems, recv_sems):
        my = lax.axis_index("i")

        xv = x_ref[...].astype(jnp.bfloat16)
        wv = w_ref[...].astype(jnp.bfloat16)
        acc = jnp.dot(xv, wv, preferred_element_type=jnp.float32)
        s = sx_ref[0] * sw_ref[0]
        yv = acc * s
        y_ref[...] = yv * jax.nn.sigmoid(yv)

        out_ref[pl.ds(my * m_per, m_per), :] = y_ref[:, pl.ds(my * n_per, n_per)]

        sends = []
        for t in range(1, N_DEV):
            j = lax.rem(my + t, N_DEV)
            rdma = pltpu.make_async_remote_copy(
                src_ref=y_ref.at[:, pl.ds(j * n_per, n_per)],
                dst_ref=out_ref.at[pl.ds(my * m_per, m_per), :],
                send_sem=send_sems.at[t],
                recv_sem=recv_sems.at[t],
                device_id=(j,),
                device_id_type=pl.DeviceIdType.MESH,
            )
            rdma.start()
            sends.append(rdma)

        for t in range(1, N_DEV):
            src_dev = lax.rem(my - t + N_DEV, N_DEV)
            recv = pltpu.make_async_remote_copy(
                src_ref=y_ref.at[:, pl.ds(src_dev * n_per, n_per)],
                dst_ref=out_ref.at[pl.ds(src_dev * m_per, m_per), :],
                send_sem=send_sems.at[t],
                recv_sem=recv_sems.at[t],
                device_id=(src_dev,),
                device_id_type=pl.DeviceIdType.MESH,
            )
            recv.wait_recv()
        for rdma in sends:
            rdma.wait_send()

    out_shape = jax.ShapeDtypeStruct((N_DEV * m_per, n_per), jnp.float32)
    return pl.pallas_call(
        body,
        out_shape=out_shape,
        in_specs=[
            pl.BlockSpec(memory_space=pltpu.VMEM),
            pl.BlockSpec(memory_space=pltpu.VMEM),
            pl.BlockSpec(memory_space=pltpu.SMEM),
            pl.BlockSpec(memory_space=pltpu.SMEM),
        ],
        out_specs=pl.BlockSpec(memory_space=pltpu.VMEM),
        scratch_shapes=[
            pltpu.VMEM((m_per, n), jnp.float32),
            pltpu.SemaphoreType.DMA((N_DEV,)),
            pltpu.SemaphoreType.DMA((N_DEV,)),
        ],
        compiler_params=pltpu.CompilerParams(collective_id=0),
    )(x, w_mat, scale_x, scale_w)


# baseline (device time: 54697 ns/iter reference)
import jax
import jax.numpy as jnp
from jax import lax
from jax.experimental import pallas as pl
from jax.experimental.pallas import tpu as pltpu

N_DEV = 32


def kernel(x, w_mat, scale_x, scale_w):
    m_per, k = x.shape
    _, n = w_mat.shape
    n_per = n // N_DEV

    def body(x_ref, w_ref, sx_ref, sw_ref, out_ref, comm_ref, send_sems, recv_sems):
        my = lax.axis_index("i")

        xv = x_ref[...].astype(jnp.bfloat16)
        wv = w_ref[...].astype(jnp.bfloat16)
        acc = jnp.dot(xv, wv, preferred_element_type=jnp.float32)
        s = sx_ref[0] * sw_ref[0]
        yv = acc * s
        yv = yv * jax.nn.sigmoid(yv)

        for j in range(N_DEV):
            comm_ref[j] = yv[:, j * n_per:(j + 1) * n_per]

        my_row = pl.multiple_of(my * m_per, 128)

        out_ref[pl.ds(my_row, m_per), :] = comm_ref[my]

        sends = []
        for t in range(1, N_DEV):
            j = lax.rem(my + t, N_DEV)
            rdma = pltpu.make_async_remote_copy(
                src_ref=comm_ref.at[j],
                dst_ref=out_ref.at[pl.ds(my_row, m_per), :],
                send_sem=send_sems.at[t],
                recv_sem=recv_sems.at[t],
                device_id=(j,),
                device_id_type=pl.DeviceIdType.MESH,
            )
            rdma.start()
            sends.append(rdma)

        for t in range(1, N_DEV):
            src_dev = lax.rem(my - t + N_DEV, N_DEV)
            src_row = pl.multiple_of(src_dev * m_per, 128)
            recv = pltpu.make_async_remote_copy(
                src_ref=comm_ref.at[src_dev],
                dst_ref=out_ref.at[pl.ds(src_row, m_per), :],
                send_sem=send_sems.at[t],
                recv_sem=recv_sems.at[t],
                device_id=(src_dev,),
                device_id_type=pl.DeviceIdType.MESH,
            )
            recv.wait_recv()
        for rdma in sends:
            rdma.wait_send()

    out_shape = jax.ShapeDtypeStruct((N_DEV * m_per, n_per), jnp.float32)
    return pl.pallas_call(
        body,
        out_shape=out_shape,
        in_specs=[
            pl.BlockSpec(memory_space=pltpu.VMEM),
            pl.BlockSpec(memory_space=pltpu.VMEM),
            pl.BlockSpec(memory_space=pltpu.SMEM),
            pl.BlockSpec(memory_space=pltpu.SMEM),
        ],
        out_specs=pl.BlockSpec(memory_space=pltpu.VMEM),
        scratch_shapes=[
            pltpu.VMEM((N_DEV, m_per, n_per), jnp.float32),
            pltpu.SemaphoreType.DMA((N_DEV,)),
            pltpu.SemaphoreType.DMA((N_DEV,)),
        ],
        compiler_params=pltpu.CompilerParams(
            vmem_limit_bytes=100 * 1024 * 1024,
        ),
    )(x, w_mat, scale_x, scale_w)


# device time: 26997 ns/iter; 2.0260x vs baseline; 2.0260x over previous
import jax
import jax.numpy as jnp
from jax import lax
from jax.experimental import pallas as pl
from jax.experimental.pallas import tpu as pltpu

N_DEV = 32
N_CHUNKS = 8
N_SLOTS = 4


def kernel(x, w_mat, scale_x, scale_w):
    m_per, k = x.shape
    _, n = w_mat.shape
    n_per = n // N_DEV
    n_chunk = n // N_CHUNKS
    tgt_per_chunk = N_DEV // N_CHUNKS

    def body(x_ref, w_hbm, sx_ref, sw_ref, out_ref,
             wbuf, commT, recvT, wdma_sems, send_sems, recv_sems, credit_sems):
        my = lax.axis_index("i")

        barrier_sem = pltpu.get_barrier_semaphore()
        pl.semaphore_signal(barrier_sem, inc=1)
        pl.semaphore_wait(barrier_sem, 1)

        for p in range(N_DEV):
            @pl.when(p != my)
            def _():
                pl.semaphore_signal(
                    credit_sems.at[my], inc=1,
                    device_id=(p,), device_id_type=pl.DeviceIdType.MESH,
                )

        def w_dma(c, slot):
            return pltpu.make_async_copy(
                w_hbm.at[:, pl.ds(c * n_chunk, n_chunk)],
                wbuf.at[slot],
                wdma_sems.at[slot],
            )

        for c in range(min(N_SLOTS - 1, N_CHUNKS)):
            w_dma(c, c % N_SLOTS).start()

        s = sx_ref[0] * sw_ref[0]
        xv = x_ref[...]
        sends = []
        for c in range(N_CHUNKS):
            slot = c % N_SLOTS
            w_dma(c, slot).wait()
            accT = lax.dot_general(
                wbuf[slot], xv,
                dimension_numbers=(((0,), (1,)), ((), ())),
                preferred_element_type=jnp.float32,
            )
            nxt = c + N_SLOTS - 1
            if nxt < N_CHUNKS:
                w_dma(nxt, nxt % N_SLOTS).start()
            yT = accT * s
            yT = yT * jax.nn.sigmoid(yT)
            for u in range(tgt_per_chunk):
                j = c * tgt_per_chunk + u
                commT[j] = yT[u * n_per:(u + 1) * n_per, :]
                rdma = pltpu.make_async_remote_copy(
                    src_ref=commT.at[j],
                    dst_ref=recvT.at[my],
                    send_sem=send_sems.at[j],
                    recv_sem=recv_sems.at[my],
                    device_id=(j,),
                    device_id_type=pl.DeviceIdType.MESH,
                )

                @pl.when(j != my)
                def _():
                    pl.semaphore_wait(credit_sems.at[j], 1)
                    rdma.start()

                @pl.when(j == my)
                def _():
                    recvT[j] = commT[j]

                sends.append((j, rdma))

        for u in range(N_DEV):
            @pl.when(u != my)
            def _():
                recv = pltpu.make_async_remote_copy(
                    src_ref=commT.at[u],
                    dst_ref=recvT.at[u],
                    send_sem=send_sems.at[u],
                    recv_sem=recv_sems.at[u],
                    device_id=(u,),
                    device_id_type=pl.DeviceIdType.MESH,
                )
                recv.wait_recv()
            out_ref[u * m_per:(u + 1) * m_per, :] = jnp.swapaxes(recvT[u], 0, 1)

        for j, rdma in sends:
            @pl.when(j != my)
            def _():
                rdma.wait_send()

    out_shape = jax.ShapeDtypeStruct((N_DEV * m_per, n_per), jnp.float32)
    return pl.pallas_call(
        body,
        out_shape=out_shape,
        in_specs=[
            pl.BlockSpec(memory_space=pltpu.VMEM),
            pl.BlockSpec(memory_space=pltpu.MemorySpace.HBM),
            pl.BlockSpec(memory_space=pltpu.SMEM),
            pl.BlockSpec(memory_space=pltpu.SMEM),
        ],
        out_specs=pl.BlockSpec(memory_space=pltpu.VMEM),
        scratch_shapes=[
            pltpu.VMEM((N_SLOTS, k, n_chunk), jnp.float32),
            pltpu.VMEM((N_DEV, n_per, m_per), jnp.float32),
            pltpu.VMEM((N_DEV, n_per, m_per), jnp.float32),
            pltpu.SemaphoreType.DMA((N_SLOTS,)),
            pltpu.SemaphoreType.DMA((N_DEV,)),
            pltpu.SemaphoreType.DMA((N_DEV,)),
            pltpu.SemaphoreType.REGULAR((N_DEV,)),
        ],
        compiler_params=pltpu.CompilerParams(
            vmem_limit_bytes=100 * 1024 * 1024,
            collective_id=0,
        ),
    )(x, w_mat, scale_x, scale_w)


# device time: 22637 ns/iter; 2.4163x vs baseline; 1.1926x over previous
import jax
import jax.numpy as jnp
from jax import lax
from jax.experimental import pallas as pl
from jax.experimental.pallas import tpu as pltpu

N_DEV = 32
N_CHUNKS = 8
N_SLOTS = 4


def kernel(x, w_mat, scale_x, scale_w):
    m_per, k = x.shape
    _, n = w_mat.shape
    n_per = n // N_DEV
    n_chunk = n // N_CHUNKS
    tgt_per_chunk = N_DEV // N_CHUNKS

    def body(x_ref, w_hbm, sx_ref, sw_ref, out_ref,
             wbuf, commT, recvT, wdma_sems, send_sems, recv_sems, credit_sems):
        my = lax.axis_index("i")

        barrier_sem = pltpu.get_barrier_semaphore()
        pl.semaphore_signal(barrier_sem, inc=1)
        pl.semaphore_wait(barrier_sem, 1)

        for p in range(N_DEV):
            @pl.when(p != my)
            def _():
                pl.semaphore_signal(
                    credit_sems.at[my], inc=1,
                    device_id=(p,), device_id_type=pl.DeviceIdType.MESH,
                )

        def w_dma(c, slot):
            return pltpu.make_async_copy(
                w_hbm.at[:, pl.ds(c * n_chunk, n_chunk)],
                wbuf.at[slot],
                wdma_sems.at[slot],
            )

        for c in range(min(N_SLOTS - 1, N_CHUNKS)):
            w_dma(c, c % N_SLOTS).start()

        s = sx_ref[0] * sw_ref[0]
        xv = x_ref[...]
        sends = []
        for c in range(N_CHUNKS):
            slot = c % N_SLOTS
            w_dma(c, slot).wait()
            accT = lax.dot_general(
                wbuf[slot], xv,
                dimension_numbers=(((0,), (1,)), ((), ())),
                preferred_element_type=jnp.float32,
            )
            nxt = c + N_SLOTS - 1
            if nxt < N_CHUNKS:
                w_dma(nxt, nxt % N_SLOTS).start()
            yT = accT * s
            yT = yT * jax.nn.sigmoid(yT)
            yTb = yT.astype(jnp.bfloat16)
            for u in range(tgt_per_chunk):
                j = c * tgt_per_chunk + u
                commT[j] = yTb[u * n_per:(u + 1) * n_per, :]
                rdma = pltpu.make_async_remote_copy(
                    src_ref=commT.at[j],
                    dst_ref=recvT.at[my],
                    send_sem=send_sems.at[j],
                    recv_sem=recv_sems.at[my],
                    device_id=(j,),
                    device_id_type=pl.DeviceIdType.MESH,
                )

                @pl.when(j != my)
                def _():
                    pl.semaphore_wait(credit_sems.at[j], 1)
                    rdma.start()

                @pl.when(j == my)
                def _():
                    recvT[j] = commT[j]

                sends.append((j, rdma))

        for u in range(N_DEV):
            @pl.when(u != my)
            def _():
                recv = pltpu.make_async_remote_copy(
                    src_ref=commT.at[u],
                    dst_ref=recvT.at[u],
                    send_sem=send_sems.at[u],
                    recv_sem=recv_sems.at[u],
                    device_id=(u,),
                    device_id_type=pl.DeviceIdType.MESH,
                )
                recv.wait_recv()
            out_ref[u * m_per:(u + 1) * m_per, :] = jnp.swapaxes(
                recvT[u], 0, 1).astype(jnp.float32)

        for j, rdma in sends:
            @pl.when(j != my)
            def _():
                rdma.wait_send()

    out_shape = jax.ShapeDtypeStruct((N_DEV * m_per, n_per), jnp.float32)
    return pl.pallas_call(
        body,
        out_shape=out_shape,
        in_specs=[
            pl.BlockSpec(memory_space=pltpu.VMEM),
            pl.BlockSpec(memory_space=pltpu.MemorySpace.HBM),
            pl.BlockSpec(memory_space=pltpu.SMEM),
            pl.BlockSpec(memory_space=pltpu.SMEM),
        ],
        out_specs=pl.BlockSpec(memory_space=pltpu.VMEM),
        scratch_shapes=[
            pltpu.VMEM((N_SLOTS, k, n_chunk), jnp.float32),
            pltpu.VMEM((N_DEV, n_per, m_per), jnp.bfloat16),
            pltpu.VMEM((N_DEV, n_per, m_per), jnp.bfloat16),
            pltpu.SemaphoreType.DMA((N_SLOTS,)),
            pltpu.SemaphoreType.DMA((N_DEV,)),
            pltpu.SemaphoreType.DMA((N_DEV,)),
            pltpu.SemaphoreType.REGULAR((N_DEV,)),
        ],
        compiler_params=pltpu.CompilerParams(
            vmem_limit_bytes=100 * 1024 * 1024,
            collective_id=0,
        ),
    )(x, w_mat, scale_x, scale_w)
